# Initial kernel scaffold; baseline (speedup 1.0000x reference)
import jax
import jax.numpy as jnp
from jax.experimental import pallas as pl


def kernel(x, A, W, temperature):
    raise NotImplementedError("write your pallas kernel here")


# speedup vs baseline: 18.1469x; 18.1469x over previous
import jax
import jax.numpy as jnp
from jax.experimental import pallas as pl
import jax.experimental.pallas.tpu as pltpu

B, N, D, K = 8, 2048, 128, 10
BN = 256
NB = N // BN
_BIG = 3.0e38


def _knn_kernel(x_ref, w_ref, t_ref, xe_ref, src_ref, tgt_ref, lp_ref,
                xe_s, sqc_s, sqr_s):
    b = pl.program_id(0)
    i = pl.program_id(1)

    @pl.when(i == 0)
    def _():
        xe = jnp.dot(x_ref[0], w_ref[:, :], preferred_element_type=jnp.float32)
        xe_s[:, :] = xe
        sq = jnp.sum(xe * xe, axis=1)
        sqc_s[:, 0] = sq
        sqr_s[0, :] = sq

    t = jnp.exp(jnp.clip(t_ref[0, 0], -5.0, 5.0))
    xq = xe_s[pl.ds(i * BN, BN), :]
    xe_ref[0, :, :] = xq

    gram_t = jax.lax.dot_general(
        xe_s[:, :], xq, (((1,), (1,)), ((), ())),
        preferred_element_type=jnp.float32)
    sqq = sqr_s[0, pl.ds(i * BN, BN)]
    d2 = sqc_s[:, :] + sqq[None, :] - 2.0 * gram_t
    vals = jnp.maximum(d2, 0.0) * t

    iota = jax.lax.broadcasted_iota(
        jnp.int32, (N, BN), 0).astype(jnp.float32)
    idx_rows = []
    val_rows = []
    for _ in range(K):
        m = jnp.min(vals, axis=0, keepdims=True)
        idx = jnp.min(jnp.where(vals == m, iota, jnp.float32(N)),
                      axis=0, keepdims=True)
        val_rows.append(m)
        idx_rows.append(idx)
        vals = jnp.where(iota == idx, jnp.float32(_BIG), vals)
    idx_t = jnp.concatenate(idx_rows, axis=0).astype(jnp.int32)
    lp_t = jnp.concatenate(val_rows, axis=0)

    lp_ref[0, :, :] = lp_t.T
    rows = (b * N + i * BN) + jax.lax.broadcasted_iota(jnp.int32, (BN, K), 0)
    src_ref[0, :, :] = rows
    tgt_ref[0, :, :] = idx_t.T + b * N


@jax.jit
def kernel(x, A, W, temperature):
    del A
    temp = temperature.reshape(1, 1).astype(jnp.float32)
    grid = (B, NB)
    xe, src, tgt, lp = pl.pallas_call(
        _knn_kernel,
        grid=grid,
        in_specs=[
            pl.BlockSpec((1, N, D), lambda b, i: (b, 0, 0)),
            pl.BlockSpec((D, D), lambda b, i: (0, 0)),
            pl.BlockSpec(memory_space=pltpu.SMEM),
        ],
        out_specs=[
            pl.BlockSpec((1, BN, D), lambda b, i: (b, i, 0)),
            pl.BlockSpec((1, BN, K), lambda b, i: (b, i, 0)),
            pl.BlockSpec((1, BN, K), lambda b, i: (b, i, 0)),
            pl.BlockSpec((1, BN, K), lambda b, i: (b, i, 0)),
        ],
        out_shape=[
            jax.ShapeDtypeStruct((B, N, D), jnp.float32),
            jax.ShapeDtypeStruct((B, N, K), jnp.int32),
            jax.ShapeDtypeStruct((B, N, K), jnp.int32),
            jax.ShapeDtypeStruct((B, N, K), jnp.float32),
        ],
        scratch_shapes=[
            pltpu.VMEM((N, D), jnp.float32),
            pltpu.VMEM((N, 1), jnp.float32),
            pltpu.VMEM((1, N), jnp.float32),
        ],
    )(x, W, temp)
    edges = jnp.concatenate(
        [src.reshape(1, -1), tgt.reshape(1, -1)], axis=0)
    return (xe, edges, lp)


# speedup vs baseline: 18.2601x; 1.0062x over previous
import jax
import jax.numpy as jnp
from jax.experimental import pallas as pl
import jax.experimental.pallas.tpu as pltpu

B, N, D, K = 8, 2048, 128, 10
BN = 256
NB = N // BN
_BIG = 3.0e38


def _knn_kernel(x_ref, w_ref, t_ref, xe_ref, src_ref, tgt_ref, lp_ref,
                xe_s, sqc_s, sqr_s):
    b = pl.program_id(0)
    i = pl.program_id(1)

    @pl.when(i == 0)
    def _():
        xe = jnp.dot(x_ref[0], w_ref[:, :], preferred_element_type=jnp.float32)
        xe_s[:, :] = xe
        sq = jnp.sum(xe * xe, axis=1)
        sqc_s[:, 0] = sq
        sqr_s[0, :] = sq

    t = jnp.exp(jnp.clip(t_ref[0, 0], -5.0, 5.0))
    xq = xe_s[pl.ds(i * BN, BN), :]
    xe_ref[0, :, :] = xq

    gram_t = jax.lax.dot_general(
        xe_s[:, :], xq, (((1,), (1,)), ((), ())),
        preferred_element_type=jnp.float32)
    vals = sqc_s[:, :] - 2.0 * gram_t

    iota = jax.lax.broadcasted_iota(
        jnp.int32, (N, BN), 0).astype(jnp.float32)
    idx_rows = []
    val_rows = []
    for _ in range(K):
        m = jnp.min(vals, axis=0, keepdims=True)
        idx = jnp.min(jnp.where(vals == m, iota, jnp.float32(N)),
                      axis=0, keepdims=True)
        val_rows.append(m)
        idx_rows.append(idx)
        vals = jnp.where(iota == idx, jnp.float32(_BIG), vals)
    idx_t = jnp.concatenate(idx_rows, axis=0).astype(jnp.int32)
    sqq = sqr_s[0, pl.ds(i * BN, BN)]
    u_t = jnp.concatenate(val_rows, axis=0)
    lp_t = jnp.maximum(u_t + sqq[None, :], 0.0) * t

    lp_ref[0, :, :] = lp_t.T
    rows = (b * N + i * BN) + jax.lax.broadcasted_iota(jnp.int32, (BN, K), 0)
    src_ref[0, :, :] = rows
    tgt_ref[0, :, :] = idx_t.T + b * N


@jax.jit
def kernel(x, A, W, temperature):
    del A
    temp = temperature.reshape(1, 1).astype(jnp.float32)
    grid = (B, NB)
    xe, src, tgt, lp = pl.pallas_call(
        _knn_kernel,
        grid=grid,
        in_specs=[
            pl.BlockSpec((1, N, D), lambda b, i: (b, 0, 0)),
            pl.BlockSpec((D, D), lambda b, i: (0, 0)),
            pl.BlockSpec(memory_space=pltpu.SMEM),
        ],
        out_specs=[
            pl.BlockSpec((1, BN, D), lambda b, i: (b, i, 0)),
            pl.BlockSpec((1, BN, K), lambda b, i: (b, i, 0)),
            pl.BlockSpec((1, BN, K), lambda b, i: (b, i, 0)),
            pl.BlockSpec((1, BN, K), lambda b, i: (b, i, 0)),
        ],
        out_shape=[
            jax.ShapeDtypeStruct((B, N, D), jnp.float32),
            jax.ShapeDtypeStruct((B, N, K), jnp.int32),
            jax.ShapeDtypeStruct((B, N, K), jnp.int32),
            jax.ShapeDtypeStruct((B, N, K), jnp.float32),
        ],
        scratch_shapes=[
            pltpu.VMEM((N, D), jnp.float32),
            pltpu.VMEM((N, 1), jnp.float32),
            pltpu.VMEM((1, N), jnp.float32),
        ],
    )(x, W, temp)
    edges = jnp.concatenate(
        [src.reshape(1, -1), tgt.reshape(1, -1)], axis=0)
    return (xe, edges, lp)


# speedup vs baseline: 19.8906x; 1.0893x over previous
import jax
import jax.numpy as jnp
from jax.experimental import pallas as pl
import jax.experimental.pallas.tpu as pltpu

B, N, D, K = 8, 2048, 128, 10
BN = 256
NB = N // BN
_BIG = 3.0e38


def _knn_kernel(x_ref, w_ref, t_ref, xe_ref, src_ref, tgt_ref, lp_ref,
                xe_s, sqc_s, sqr_s):
    b = pl.program_id(0)
    i = pl.program_id(1)

    @pl.when(i == 0)
    def _():
        xe = jnp.dot(x_ref[0], w_ref[:, :], preferred_element_type=jnp.float32)
        xe_s[:, :] = xe
        sq = jnp.sum(xe * xe, axis=1)
        sqc_s[:, 0] = sq
        sqr_s[0, :] = sq

    t = jnp.exp(jnp.clip(t_ref[0, 0], -5.0, 5.0))
    xq = xe_s[pl.ds(i * BN, BN), :]
    xe_ref[0, :, :] = xq

    gram_t = jax.lax.dot_general(
        xe_s[:, :], xq, (((1,), (1,)), ((), ())),
        preferred_element_type=jnp.float32)
    iota = jax.lax.broadcasted_iota(
        jnp.int32, (N, BN), 0).astype(jnp.float32)
    self_f = (jax.lax.broadcasted_iota(jnp.int32, (1, BN), 1)
              + i * BN).astype(jnp.float32)
    vals = jnp.where(iota == self_f, jnp.float32(_BIG),
                     sqc_s[:, :] - 2.0 * gram_t)

    idx_rows = [self_f]
    val_rows = []
    for _ in range(K - 1):
        m = jnp.min(vals, axis=0, keepdims=True)
        idx = jnp.min(jnp.where(vals == m, iota, jnp.float32(N)),
                      axis=0, keepdims=True)
        val_rows.append(m)
        idx_rows.append(idx)
        vals = jnp.where(iota == idx, jnp.float32(_BIG), vals)
    idx_t = jnp.concatenate(idx_rows, axis=0).astype(jnp.int32)
    sqq = sqr_s[0, pl.ds(i * BN, BN)]
    u_t = jnp.concatenate(val_rows, axis=0)
    lp_t = jnp.concatenate(
        [jnp.zeros((1, BN), jnp.float32),
         jnp.maximum(u_t + sqq[None, :], 0.0) * t], axis=0)

    lp_ref[0, :, :] = lp_t.T
    rows = (b * N + i * BN) + jax.lax.broadcasted_iota(jnp.int32, (BN, K), 0)
    src_ref[0, :, :] = rows
    tgt_ref[0, :, :] = idx_t.T + b * N


@jax.jit
def kernel(x, A, W, temperature):
    del A
    temp = temperature.reshape(1, 1).astype(jnp.float32)
    grid = (B, NB)
    xe, src, tgt, lp = pl.pallas_call(
        _knn_kernel,
        grid=grid,
        in_specs=[
            pl.BlockSpec((1, N, D), lambda b, i: (b, 0, 0)),
            pl.BlockSpec((D, D), lambda b, i: (0, 0)),
            pl.BlockSpec(memory_space=pltpu.SMEM),
        ],
        out_specs=[
            pl.BlockSpec((1, BN, D), lambda b, i: (b, i, 0)),
            pl.BlockSpec((1, BN, K), lambda b, i: (b, i, 0)),
            pl.BlockSpec((1, BN, K), lambda b, i: (b, i, 0)),
            pl.BlockSpec((1, BN, K), lambda b, i: (b, i, 0)),
        ],
        out_shape=[
            jax.ShapeDtypeStruct((B, N, D), jnp.float32),
            jax.ShapeDtypeStruct((B, N, K), jnp.int32),
            jax.ShapeDtypeStruct((B, N, K), jnp.int32),
            jax.ShapeDtypeStruct((B, N, K), jnp.float32),
        ],
        scratch_shapes=[
            pltpu.VMEM((N, D), jnp.float32),
            pltpu.VMEM((N, 1), jnp.float32),
            pltpu.VMEM((1, N), jnp.float32),
        ],
    )(x, W, temp)
    edges = jnp.concatenate(
        [src.reshape(1, -1), tgt.reshape(1, -1)], axis=0)
    return (xe, edges, lp)


# speedup vs baseline: 27.3991x; 1.3775x over previous
import jax
import jax.numpy as jnp
from jax.experimental import pallas as pl
import jax.experimental.pallas.tpu as pltpu

B, N, D, K = 8, 2048, 128, 10
BN = 512
NB = N // BN
_BIG = 3.0e38


def _knn_kernel(x_ref, w_ref, t_ref, xe_ref, src_ref, tgt_ref, lp_ref,
                xe_s, sqc_s, sqr_s):
    b = pl.program_id(0)
    i = pl.program_id(1)

    @pl.when(i == 0)
    def _():
        xe = jnp.dot(x_ref[0], w_ref[:, :], preferred_element_type=jnp.float32)
        xe_s[:, :] = xe
        sq = jnp.sum(xe * xe, axis=1)
        sqc_s[:, 0] = sq
        sqr_s[0, :] = sq

    t = jnp.exp(jnp.clip(t_ref[0, 0], -5.0, 5.0))
    xq = xe_s[pl.ds(i * BN, BN), :]
    xe_ref[0, :, :] = xq

    gram_t = jax.lax.dot_general(
        xe_s[:, :], xq, (((1,), (1,)), ((), ())),
        preferred_element_type=jnp.float32)
    iota = jax.lax.broadcasted_iota(
        jnp.int32, (N, BN), 0).astype(jnp.float32)
    self_f = (jax.lax.broadcasted_iota(jnp.int32, (1, BN), 1)
              + i * BN).astype(jnp.float32)
    vals = jnp.where(iota == self_f, jnp.float32(_BIG),
                     sqc_s[:, :] - 2.0 * gram_t)

    idx_rows = [self_f]
    val_rows = []
    for _ in range(K - 1):
        m = jnp.min(vals, axis=0, keepdims=True)
        idx = jnp.min(jnp.where(vals == m, iota, jnp.float32(N)),
                      axis=0, keepdims=True)
        val_rows.append(m)
        idx_rows.append(idx)
        vals = jnp.where(iota == idx, jnp.float32(_BIG), vals)
    idx_t = jnp.concatenate(idx_rows, axis=0).astype(jnp.int32)
    sqq = sqr_s[0, pl.ds(i * BN, BN)]
    u_t = jnp.concatenate(val_rows, axis=0)
    lp_t = jnp.concatenate(
        [jnp.zeros((1, BN), jnp.float32),
         jnp.maximum(u_t + sqq[None, :], 0.0) * t], axis=0)

    lp_ref[0, :, :] = lp_t.T
    rows = (b * N + i * BN) + jax.lax.broadcasted_iota(jnp.int32, (BN, K), 0)
    src_ref[0, :, :] = rows
    tgt_ref[0, :, :] = idx_t.T + b * N


@jax.jit
def kernel(x, A, W, temperature):
    del A
    temp = temperature.reshape(1, 1).astype(jnp.float32)
    grid = (B, NB)
    xe, src, tgt, lp = pl.pallas_call(
        _knn_kernel,
        grid=grid,
        in_specs=[
            pl.BlockSpec((1, N, D), lambda b, i: (b, 0, 0)),
            pl.BlockSpec((D, D), lambda b, i: (0, 0)),
            pl.BlockSpec(memory_space=pltpu.SMEM),
        ],
        out_specs=[
            pl.BlockSpec((1, BN, D), lambda b, i: (b, i, 0)),
            pl.BlockSpec((1, BN, K), lambda b, i: (b, i, 0)),
            pl.BlockSpec((1, BN, K), lambda b, i: (b, i, 0)),
            pl.BlockSpec((1, BN, K), lambda b, i: (b, i, 0)),
        ],
        out_shape=[
            jax.ShapeDtypeStruct((B, N, D), jnp.float32),
            jax.ShapeDtypeStruct((B, N, K), jnp.int32),
            jax.ShapeDtypeStruct((B, N, K), jnp.int32),
            jax.ShapeDtypeStruct((B, N, K), jnp.float32),
        ],
        scratch_shapes=[
            pltpu.VMEM((N, D), jnp.float32),
            pltpu.VMEM((N, 1), jnp.float32),
            pltpu.VMEM((1, N), jnp.float32),
        ],
    )(x, W, temp)
    edges = jnp.concatenate(
        [src.reshape(1, -1), tgt.reshape(1, -1)], axis=0)
    return (xe, edges, lp)


# speedup vs baseline: 28.7731x; 1.0501x over previous
import jax
import jax.numpy as jnp
from jax.experimental import pallas as pl
import jax.experimental.pallas.tpu as pltpu

B, N, D, K = 8, 2048, 128, 10
BN = 1024
NB = N // BN
_BIG = 3.0e38


def _knn_kernel(x_ref, w_ref, t_ref, xe_ref, src_ref, tgt_ref, lp_ref,
                xe_s, sqc_s, sqr_s):
    b = pl.program_id(0)
    i = pl.program_id(1)

    @pl.when(i == 0)
    def _():
        xe = jnp.dot(x_ref[0], w_ref[:, :], preferred_element_type=jnp.float32)
        xe_s[:, :] = xe
        sq = jnp.sum(xe * xe, axis=1)
        sqc_s[:, 0] = sq
        sqr_s[0, :] = sq

    t = jnp.exp(jnp.clip(t_ref[0, 0], -5.0, 5.0))
    xq = xe_s[pl.ds(i * BN, BN), :]
    xe_ref[0, :, :] = xq

    gram_t = jax.lax.dot_general(
        xe_s[:, :], xq, (((1,), (1,)), ((), ())),
        preferred_element_type=jnp.float32)
    iota = jax.lax.broadcasted_iota(
        jnp.int32, (N, BN), 0).astype(jnp.float32)
    self_f = (jax.lax.broadcasted_iota(jnp.int32, (1, BN), 1)
              + i * BN).astype(jnp.float32)
    vals = jnp.where(iota == self_f, jnp.float32(_BIG),
                     sqc_s[:, :] - 2.0 * gram_t)

    idx_rows = [self_f]
    val_rows = []
    for _ in range(K - 1):
        m = jnp.min(vals, axis=0, keepdims=True)
        idx = jnp.min(jnp.where(vals == m, iota, jnp.float32(N)),
                      axis=0, keepdims=True)
        val_rows.append(m)
        idx_rows.append(idx)
        vals = jnp.where(iota == idx, jnp.float32(_BIG), vals)
    idx_t = jnp.concatenate(idx_rows, axis=0).astype(jnp.int32)
    sqq = sqr_s[0, pl.ds(i * BN, BN)]
    u_t = jnp.concatenate(val_rows, axis=0)
    lp_t = jnp.concatenate(
        [jnp.zeros((1, BN), jnp.float32),
         jnp.maximum(u_t + sqq[None, :], 0.0) * t], axis=0)

    lp_ref[0, :, :] = lp_t.T
    rows = (b * N + i * BN) + jax.lax.broadcasted_iota(jnp.int32, (BN, K), 0)
    src_ref[0, :, :] = rows
    tgt_ref[0, :, :] = idx_t.T + b * N


@jax.jit
def kernel(x, A, W, temperature):
    del A
    temp = temperature.reshape(1, 1).astype(jnp.float32)
    grid = (B, NB)
    xe, src, tgt, lp = pl.pallas_call(
        _knn_kernel,
        grid=grid,
        in_specs=[
            pl.BlockSpec((1, N, D), lambda b, i: (b, 0, 0)),
            pl.BlockSpec((D, D), lambda b, i: (0, 0)),
            pl.BlockSpec(memory_space=pltpu.SMEM),
        ],
        out_specs=[
            pl.BlockSpec((1, BN, D), lambda b, i: (b, i, 0)),
            pl.BlockSpec((1, BN, K), lambda b, i: (b, i, 0)),
            pl.BlockSpec((1, BN, K), lambda b, i: (b, i, 0)),
            pl.BlockSpec((1, BN, K), lambda b, i: (b, i, 0)),
        ],
        out_shape=[
            jax.ShapeDtypeStruct((B, N, D), jnp.float32),
            jax.ShapeDtypeStruct((B, N, K), jnp.int32),
            jax.ShapeDtypeStruct((B, N, K), jnp.int32),
            jax.ShapeDtypeStruct((B, N, K), jnp.float32),
        ],
        scratch_shapes=[
            pltpu.VMEM((N, D), jnp.float32),
            pltpu.VMEM((N, 1), jnp.float32),
            pltpu.VMEM((1, N), jnp.float32),
        ],
    )(x, W, temp)
    edges = jnp.concatenate(
        [src.reshape(1, -1), tgt.reshape(1, -1)], axis=0)
    return (xe, edges, lp)
